# Initial kernel scaffold; baseline (speedup 1.0000x reference)
import jax
import jax.numpy as jnp
from jax.experimental import pallas as pl


def kernel(scores):
    raise NotImplementedError("write your pallas kernel here")


# speedup vs baseline: 10.3254x; 10.3254x over previous
import jax
import jax.numpy as jnp
from jax import lax
from jax.experimental import pallas as pl
from jax.experimental.pallas import tpu as pltpu

_K = 32
_NEG = -1e30


def _lae(a, b):
    m = jnp.maximum(a, b)
    d = jnp.abs(a - b)
    return m + jnp.log1p(jnp.exp(-d))


def _make_dp_kernel(k):
    def body(theta_nr_ref, flat_ref, g_ref, marg_ref, mask_ref,
             abstore, mcomb, xwork):
        N, R = theta_nr_ref.shape
        H = N // 2
        neg_row = jnp.full((1, R), _NEG, jnp.float32)
        rows = lax.broadcasted_iota(jnp.int32, (k + 1, R), 0)
        f0 = jnp.where(rows == 0, 0.0, _NEG).astype(jnp.float32)
        g0 = jnp.where(rows == k, 0.0, _NEG).astype(jnp.float32)

        def _lse_cols(comb):
            m = jnp.max(comb, axis=0, keepdims=True)
            s = jnp.sum(jnp.exp(comb - m), axis=0, keepdims=True)
            return m + jnp.log(s)

        def step(t, carry):
            F, Gp = carry
            j = N - 1 - t
            th_t = theta_nr_ref[pl.ds(t, 1), :]
            th_j = theta_nr_ref[pl.ds(j, 1), :]
            Gslice = Gp[1:k + 1]

            @pl.when(t < H)
            def _():
                slab = jnp.concatenate([F[:k], Gslice], axis=1)
                abstore[pl.ds(t, 1), :, :] = slab.reshape(1, k, 2 * R)

            @pl.when(t >= H)
            def _():
                slab = abstore[pl.ds(j, 1), :, :].reshape(k, 2 * R)
                x = jnp.concatenate([Gslice, F[:k]], axis=1)
                lse = _lse_cols(slab + x)
                mcomb[pl.ds(j, 1), :] = lse[:, :R]
                mcomb[pl.ds(t, 1), :] = lse[:, R:]

            Fsh = jnp.concatenate([neg_row, F[:k]], axis=0) + th_t
            Gsh = jnp.concatenate([Gp[1:], neg_row], axis=0) + th_j
            return _lae(F, Fsh), _lae(Gp, Gsh)

        F, _ = lax.fori_loop(0, N, step, (f0, g0))
        logZ = F[k:k + 1, :]
        marg_ref[:, :] = jnp.exp(theta_nr_ref[:, :] + mcomb[:, :] - logZ)

        xwork[:, :] = flat_ref[:, :] + g_ref[:, :]
        mask_ref[:, :] = jnp.zeros((R, N), jnp.float32)
        iota = lax.broadcasted_iota(jnp.int32, (R, N), 1)

        def tk_step(i, _):
            x = xwork[:, :]
            m = jnp.max(x, axis=1, keepdims=True)
            cand = x == m
            idx = jnp.min(jnp.where(cand, iota, N), axis=1, keepdims=True)
            sel = iota == idx
            mask_ref[:, :] = jnp.where(sel, 1.0, mask_ref[:, :])
            xwork[:, :] = jnp.where(sel, _NEG, x)
            return 0

        lax.fori_loop(0, k, tk_step, 0)

    return body


def _run(scores, k):
    bsz, Nmax, ens = scores.shape
    flat = jnp.transpose(scores, (0, 2, 1)).reshape(bsz * ens, Nmax)
    R, N = flat.shape
    theta_nr = flat.T
    u = jax.random.uniform(jax.random.key(42), (1, R, N),
                           minval=1e-20, maxval=1.0)
    g = -jnp.log(-jnp.log(u))

    marg_nr, mask = pl.pallas_call(
        _make_dp_kernel(k),
        out_shape=[
            jax.ShapeDtypeStruct((N, R), jnp.float32),
            jax.ShapeDtypeStruct((R, N), jnp.float32),
        ],
        scratch_shapes=[
            pltpu.VMEM((N // 2, k, 2 * R), jnp.float32),
            pltpu.VMEM((N, R), jnp.float32),
            pltpu.VMEM((R, N), jnp.float32),
        ],
    )(theta_nr, flat, g[0])

    marg = marg_nr.T
    samples = lax.stop_gradient(mask[None] - marg[None]) + marg[None]
    new_mask = samples.reshape(1, bsz, ens, N).transpose(0, 1, 3, 2)
    new_marginals = marg.reshape(bsz, ens, N).transpose(0, 2, 1)
    return new_mask, new_marginals


@jax.jit
def kernel(scores):
    return _run(scores, _K)


# speedup vs baseline: 33.6062x; 3.2547x over previous
import jax
import jax.numpy as jnp
from jax import lax
from jax.experimental import pallas as pl
from jax.experimental.pallas import tpu as pltpu

_K = 32
_NEG = -1e30
_U = 8


def _lae(a, b):
    m = jnp.maximum(a, b)
    d = jnp.abs(a - b)
    return m + jnp.log1p(jnp.exp(-d))


def _make_dp_kernel(k):
    def body(theta_nr_ref, flat_ref, g_ref, marg_ref, mask_ref,
             abstore, mstore_a, mstore_b, xwork):
        N, R = theta_nr_ref.shape
        U = _U
        NO = N // U
        HO = NO // 2
        neg_row = jnp.full((1, R), _NEG, jnp.float32)
        rowi = lax.broadcasted_iota(jnp.int32, (k + 1, R), 0)
        f0 = jnp.where(rowi == 0, 0.0, _NEG).astype(jnp.float32)
        g0 = jnp.where(rowi == k, 0.0, _NEG).astype(jnp.float32)

        def _lse_cols(comb):
            m = jnp.max(comb, axis=0, keepdims=True)
            s = jnp.sum(jnp.exp(comb - m), axis=0, keepdims=True)
            return m + jnp.log(s)

        def outer(o, carry):
            F, Gp = carry
            t0 = pl.multiple_of(U * o, U)
            j0 = pl.multiple_of(N - U - U * o, U)
            th_f = theta_nr_ref[pl.ds(t0, U), :]
            th_b = theta_nr_ref[pl.ds(j0, U), :]

            As, Gs = [], []
            for s in range(U):
                As.append(F[:k])
                Gs.append(Gp[1:k + 1])
                th_t = th_f[s:s + 1, :]
                th_j = th_b[U - 1 - s:U - s, :]
                Fsh = jnp.concatenate([neg_row, F[:k]], axis=0) + th_t
                Gsh = jnp.concatenate([Gp[1:], neg_row], axis=0) + th_j
                F = _lae(F, Fsh)
                Gp = _lae(Gp, Gsh)

            @pl.when(o < HO)
            def _():
                for s in range(U):
                    slab = jnp.concatenate([As[s], Gs[s]], axis=1)
                    abstore[pl.ds(t0 + s, 1), :, :] = slab.reshape(1, k, 2 * R)

            @pl.when(o >= HO)
            def _():
                lses = []
                for s in range(U):
                    slab = abstore[pl.ds(j0 + (U - 1 - s), 1), :, :]
                    x = jnp.concatenate([Gs[s], As[s]], axis=1)
                    lses.append(_lse_cols(slab.reshape(k, 2 * R) + x))
                lse_a = jnp.concatenate(lses, axis=0)
                lse_b = jnp.concatenate(lses[::-1], axis=0)
                mstore_a[pl.ds(o - HO, 1), :, :] = lse_a.reshape(1, U, 2 * R)
                mstore_b[pl.ds(NO - 1 - o, 1), :, :] = lse_b.reshape(1, U, 2 * R)

            return F, Gp

        F, _ = lax.fori_loop(0, NO, outer, (f0, g0))
        logZ = F[k:k + 1, :]
        mlo = mstore_b[:, :, :][:, :, :R].reshape(N // 2, R)
        mhi = mstore_a[:, :, :][:, :, R:].reshape(N // 2, R)
        mcomb = jnp.concatenate([mlo, mhi], axis=0)
        marg_ref[:, :] = jnp.exp(theta_nr_ref[:, :] + mcomb - logZ)

        xwork[:, :] = flat_ref[:, :] + g_ref[:, :]
        mask_ref[:, :] = jnp.zeros((R, N), jnp.float32)
        iota = lax.broadcasted_iota(jnp.int32, (R, N), 1)

        def tk_step(i, _):
            x = xwork[:, :]
            m = jnp.max(x, axis=1, keepdims=True)
            cand = x == m
            idx = jnp.min(jnp.where(cand, iota, N), axis=1, keepdims=True)
            sel = iota == idx
            mask_ref[:, :] = jnp.where(sel, 1.0, mask_ref[:, :])
            xwork[:, :] = jnp.where(sel, _NEG, x)
            return 0

        lax.fori_loop(0, k, tk_step, 0)

    return body


def _run(scores, k):
    bsz, Nmax, ens = scores.shape
    flat = jnp.transpose(scores, (0, 2, 1)).reshape(bsz * ens, Nmax)
    R, N = flat.shape
    theta_nr = flat.T
    u = jax.random.uniform(jax.random.key(42), (1, R, N),
                           minval=1e-20, maxval=1.0)
    g = -jnp.log(-jnp.log(u))

    marg_nr, mask = pl.pallas_call(
        _make_dp_kernel(k),
        out_shape=[
            jax.ShapeDtypeStruct((N, R), jnp.float32),
            jax.ShapeDtypeStruct((R, N), jnp.float32),
        ],
        scratch_shapes=[
            pltpu.VMEM((N // 2, k, 2 * R), jnp.float32),
            pltpu.VMEM((N // (2 * _U), _U, 2 * R), jnp.float32),
            pltpu.VMEM((N // (2 * _U), _U, 2 * R), jnp.float32),
            pltpu.VMEM((R, N), jnp.float32),
        ],
    )(theta_nr, flat, g[0])

    marg = marg_nr.T
    samples = lax.stop_gradient(mask[None] - marg[None]) + marg[None]
    new_mask = samples.reshape(1, bsz, ens, N).transpose(0, 1, 3, 2)
    new_marginals = marg.reshape(bsz, ens, N).transpose(0, 2, 1)
    return new_mask, new_marginals


@jax.jit
def kernel(scores):
    return _run(scores, _K)


# speedup vs baseline: 33.8814x; 1.0082x over previous
import jax
import jax.numpy as jnp
from jax import lax
from jax.experimental import pallas as pl
from jax.experimental.pallas import tpu as pltpu

_K = 32
_NEG = -1e30
_U = 8


def _lae(a, b):
    m = jnp.maximum(a, b)
    d = jnp.abs(a - b)
    return m + jnp.log1p(jnp.exp(-d))


def _make_dp_kernel(k):
    def body(theta_nr_ref, flat_ref, g_ref, marg_ref, mask_ref,
             abstore, mstore_a, mstore_b, xwork):
        N, R = theta_nr_ref.shape
        U = _U
        NO = N // U
        HO = NO // 2
        neg_row2 = jnp.full((1, 2 * R), _NEG, jnp.float32)
        rowi = lax.broadcasted_iota(jnp.int32, (k + 1, 2 * R), 0)
        lanei = lax.broadcasted_iota(jnp.int32, (k + 1, 2 * R), 1)
        lmask = lanei < R
        lmask_k = lmask[:k]
        fg0 = jnp.where(lmask, jnp.where(rowi == 0, 0.0, _NEG),
                        jnp.where(rowi == k, 0.0, _NEG)).astype(jnp.float32)

        def _lse_cols(comb):
            m = jnp.max(comb, axis=0, keepdims=True)
            s = jnp.sum(jnp.exp(comb - m), axis=0, keepdims=True)
            return m + jnp.log(s)

        def outer(o, FG):
            t0 = pl.multiple_of(U * o, U)
            j0 = pl.multiple_of(N - U - U * o, U)
            th_f = theta_nr_ref[pl.ds(t0, U), :]
            th_b = theta_nr_ref[pl.ds(j0, U), :]

            slabs = []
            for s in range(U):
                slabs.append(jnp.where(lmask_k, FG[:k], FG[1:k + 1]))
                th = jnp.concatenate([th_f[s:s + 1, :],
                                      th_b[U - 1 - s:U - s, :]], axis=1)
                dn = jnp.concatenate([neg_row2, FG[:k]], axis=0)
                up = jnp.concatenate([FG[1:], neg_row2], axis=0)
                FG = _lae(FG, jnp.where(lmask, dn, up) + th)

            @pl.when(o < HO)
            def _():
                for s in range(U):
                    abstore[pl.ds(t0 + s, 1), :, :] = (
                        slabs[s].reshape(1, k, 2 * R))

            @pl.when(o >= HO)
            def _():
                lses = []
                for s in range(U):
                    slab = abstore[pl.ds(j0 + (U - 1 - s), 1), :, :]
                    x = jnp.concatenate([slabs[s][:, R:], slabs[s][:, :R]],
                                        axis=1)
                    lses.append(_lse_cols(slab.reshape(k, 2 * R) + x))
                lse_a = jnp.concatenate(lses, axis=0)
                lse_b = jnp.concatenate(lses[::-1], axis=0)
                mstore_a[pl.ds(o - HO, 1), :, :] = lse_a.reshape(1, U, 2 * R)
                mstore_b[pl.ds(NO - 1 - o, 1), :, :] = lse_b.reshape(1, U, 2 * R)

            return FG

        FG = lax.fori_loop(0, NO, outer, fg0)
        logZ = FG[k:k + 1, :R]
        mlo = mstore_b[:, :, :][:, :, :R].reshape(N // 2, R)
        mhi = mstore_a[:, :, :][:, :, R:].reshape(N // 2, R)
        mcomb = jnp.concatenate([mlo, mhi], axis=0)
        marg_ref[:, :] = jnp.exp(theta_nr_ref[:, :] + mcomb - logZ)

        xwork[:, :] = flat_ref[:, :] + g_ref[:, :]
        mask_ref[:, :] = jnp.zeros((R, N), jnp.float32)
        iota = lax.broadcasted_iota(jnp.int32, (R, N), 1)

        def tk_step(i, _):
            x = xwork[:, :]
            m = jnp.max(x, axis=1, keepdims=True)
            cand = x == m
            idx = jnp.min(jnp.where(cand, iota, N), axis=1, keepdims=True)
            sel = iota == idx
            mask_ref[:, :] = jnp.where(sel, 1.0, mask_ref[:, :])
            xwork[:, :] = jnp.where(sel, _NEG, x)
            return 0

        lax.fori_loop(0, k, tk_step, 0)

    return body


def _run(scores, k):
    bsz, Nmax, ens = scores.shape
    flat = jnp.transpose(scores, (0, 2, 1)).reshape(bsz * ens, Nmax)
    R, N = flat.shape
    theta_nr = flat.T
    u = jax.random.uniform(jax.random.key(42), (1, R, N),
                           minval=1e-20, maxval=1.0)
    g = -jnp.log(-jnp.log(u))

    marg_nr, mask = pl.pallas_call(
        _make_dp_kernel(k),
        out_shape=[
            jax.ShapeDtypeStruct((N, R), jnp.float32),
            jax.ShapeDtypeStruct((R, N), jnp.float32),
        ],
        scratch_shapes=[
            pltpu.VMEM((N // 2, k, 2 * R), jnp.float32),
            pltpu.VMEM((N // (2 * _U), _U, 2 * R), jnp.float32),
            pltpu.VMEM((N // (2 * _U), _U, 2 * R), jnp.float32),
            pltpu.VMEM((R, N), jnp.float32),
        ],
    )(theta_nr, flat, g[0])

    marg = marg_nr.T
    samples = lax.stop_gradient(mask[None] - marg[None]) + marg[None]
    new_mask = samples.reshape(1, bsz, ens, N).transpose(0, 1, 3, 2)
    new_marginals = marg.reshape(bsz, ens, N).transpose(0, 2, 1)
    return new_mask, new_marginals


@jax.jit
def kernel(scores):
    return _run(scores, _K)
